# baseline (speedup 1.0000x reference)
import functools

import jax
import jax.numpy as jnp
from jax import lax
from jax.experimental import pallas as pl
from jax.experimental.pallas import tpu as pltpu
from jax.experimental.pallas import tpu_sc as plsc

L = 16
NBUF = 4
CB = 4


@functools.lru_cache(maxsize=None)
def _build(B, NCAT, NNUM, VROWS, D):
    info = plsc.get_sparse_core_info()
    NC, NS = info.num_cores, info.num_subcores
    NW = NC * NS
    NTOK = 1 + NCAT + NNUM
    KD = D // L
    RW = B // NW
    NCHUNK = RW // CB
    assert B % (NW * CB) == 0 and D % L == 0 and NCHUNK % NBUF == 0
    CW = NCAT + NNUM
    CA = NCAT + 2 * NNUM
    CROWS = -(-(CA + 1) // 8) * 8

    mesh = plsc.VectorSubcoreMesh(core_axis_name="c", subcore_axis_name="s")

    @functools.partial(
        pl.kernel,
        out_type=jax.ShapeDtypeStruct((B, NTOK, D), jnp.float32),
        mesh=mesh,
        scratch_types=(
            [pltpu.VMEM((8, 128), jnp.int32)] * NBUF
            + [pltpu.VMEM((8, 128), jnp.float32)] * NBUF
            + [pltpu.VMEM((CB, NTOK, D), jnp.float32)] * NBUF
            + [pltpu.VMEM((CROWS, 128), jnp.float32)]
            + [pltpu.SemaphoreType.DMA] * NBUF
            + [pltpu.SemaphoreType.DMA] * NBUF
            + [pltpu.SemaphoreType.DMA] * NBUF
        ),
    )
    def tokenize(blk_hbm, xblk_hbm, tab3_hbm, const_hbm, out_hbm, *refs):
        idx_v = refs[0:NBUF]
        xnum_v = refs[NBUF:2 * NBUF]
        out_v = refs[2 * NBUF:3 * NBUF]
        const_v = refs[3 * NBUF]
        isem = refs[3 * NBUF + 1:4 * NBUF + 1]
        gsem = refs[4 * NBUF + 1:5 * NBUF + 1]
        wsem = refs[5 * NBUF + 1:6 * NBUF + 1]

        wid = lax.axis_index("s") * NC + lax.axis_index("c")
        base = wid * RW
        pltpu.sync_copy(const_hbm, const_v)

        def load_inputs(g, b):
            pltpu.async_copy(blk_hbm.at[wid, g], idx_v[b], isem[b])
            pltpu.async_copy(xblk_hbm.at[wid, g], xnum_v[b], isem[b])

        def wait_inputs(g, b):
            pltpu.make_async_copy(blk_hbm.at[wid, g], idx_v[b],
                                  isem[b]).wait()
            pltpu.make_async_copy(xblk_hbm.at[wid, g], xnum_v[b],
                                  isem[b]).wait()

        def fire_gathers(b):
            def row(r, c):
                v0 = idx_v[b][r, pl.ds(0, L)]
                v1 = idx_v[b][r, pl.ds(L, L)]
                for i in range(NCAT):
                    rowid = v0[i] if i < L else v1[i - L]
                    pltpu.async_copy(
                        tab3_hbm.at[i, pl.ds(rowid, 1), :],
                        out_v[b].at[pl.ds(r, 1), 1 + i, :], gsem[b])
                return c

            lax.fori_loop(0, CB, row, 0)

        def fill_rows(b):
            def row(r, c):
                xv = xnum_v[b][r, pl.ds(0, L)]
                for k in range(KD):
                    sl = pl.ds(k * L, L)
                    out_v[b][r, 0, sl] = const_v[CA, sl]
                for j in range(NNUM):
                    x = xv[j]
                    for k in range(KD):
                        sl = pl.ds(k * L, L)
                        out_v[b][r, 1 + NCAT + j, sl] = (
                            x * const_v[NCAT + j, sl] + const_v[CW + j, sl])
                return c

            lax.fori_loop(0, CB, row, 0)

        def drain_and_addpos(b):
            def wrow(r, c):
                for i in range(NCAT):
                    pltpu.make_async_copy(
                        tab3_hbm.at[i, pl.ds(0, 1), :],
                        out_v[b].at[pl.ds(r, 1), 1 + i, :], gsem[b]).wait()
                return c

            lax.fori_loop(0, CB, wrow, 0)

            def add_pos(r, c):
                for i in range(NCAT):
                    for k in range(KD):
                        sl = pl.ds(k * L, L)
                        out_v[b][r, 1 + i, sl] = (
                            out_v[b][r, 1 + i, sl] + const_v[i, sl])
                return c

            lax.fori_loop(0, CB, add_pos, 0)

        def issue_writeback(g, b):
            b0 = base + g * CB
            pltpu.async_copy(out_v[b], out_hbm.at[pl.ds(b0, CB)], wsem[b])

        def wait_writeback(g, b):
            b0 = base + g * CB
            pltpu.make_async_copy(out_v[b], out_hbm.at[pl.ds(b0, CB)],
                                  wsem[b]).wait()

        for b in range(NBUF - 1):
            load_inputs(b, b)
        wait_inputs(0, 0)
        fire_gathers(0)

        def ring(h, carry):
            g0 = h * NBUF
            for b in range(NBUF):
                g = g0 + b
                gn = g + 1
                bn = (b + 1) % NBUF
                @pl.when(gn < NCHUNK)
                def _(gn=gn, bn=bn):
                    @pl.when(gn >= NBUF)
                    def _():
                        wait_writeback(gn - NBUF, bn)

                    wait_inputs(gn, bn)
                    fire_gathers(bn)

                fill_rows(b)
                @pl.when(g + NBUF - 1 < NCHUNK)
                def _(b=b, g=g):
                    load_inputs(g + NBUF - 1, (b + NBUF - 1) % NBUF)

                drain_and_addpos(b)
                issue_writeback(g, b)
            return carry

        lax.fori_loop(0, NCHUNK // NBUF, ring, 0)
        for b in range(NBUF):
            wait_writeback(NCHUNK - NBUF + b, b)

    return tokenize


def kernel(x_cat, x_num, cat_tables, num_w, num_b, feature_pos, cls):
    B, NCAT = x_cat.shape
    NNUM = x_num.shape[1]
    VROWS, D = cat_tables.shape[1], cat_tables.shape[2]
    NW = 32
    NCHUNK = B // (NW * CB)
    CW = NCAT + NNUM
    CA = NCAT + 2 * NNUM
    CROWS = -(-(CA + 1) // 8) * 8
    xi = x_cat.astype(jnp.int32).reshape(NW, NCHUNK, CB, NCAT)
    xf = x_num.astype(jnp.float32).reshape(NW, NCHUNK, CB, NNUM)
    blk = (jnp.zeros((NW, NCHUNK, 8, 128), jnp.int32)
           .at[:, :, :CB, :NCAT].set(xi))
    xblk = (jnp.zeros((NW, NCHUNK, 8, 128), jnp.float32)
            .at[:, :, :CB, :NNUM].set(xf))
    cpad = (jnp.zeros((CROWS, 128), jnp.float32)
            .at[:NCAT, :D].set(feature_pos[1:1 + NCAT])
            .at[NCAT:CW, :D].set(num_w)
            .at[CW:CA, :D].set(num_b + feature_pos[1 + NCAT:])
            .at[CA, :D].set(cls.reshape(D) + feature_pos[0]))
    fn = _build(B, NCAT, NNUM, VROWS, D)
    return fn(blk, xblk, cat_tables, cpad)


# speedup vs baseline: 1.1797x; 1.1797x over previous
import functools

import jax
import jax.numpy as jnp
from jax import lax
from jax.experimental import pallas as pl
from jax.experimental.pallas import tpu as pltpu
from jax.experimental.pallas import tpu_sc as plsc

L = 16
NBUF = 4
CB = 4


@functools.lru_cache(maxsize=None)
def _build_gather(B, NCAT, VROWS, D):
    info = plsc.get_sparse_core_info()
    NC, NS = info.num_cores, info.num_subcores
    NW = NC * NS
    GR = NCAT // 2
    RW = B // NW
    NCHUNK = RW // CB
    assert B % (NW * CB) == 0 and NCAT % 2 == 0 and NCHUNK % NBUF == 0

    mesh = plsc.VectorSubcoreMesh(core_axis_name="c", subcore_axis_name="s")

    @functools.partial(
        pl.kernel,
        out_type=jax.ShapeDtypeStruct((B, NCAT, D), jnp.float32),
        mesh=mesh,
        scratch_types=(
            [pltpu.VMEM((CB, 128), jnp.int32)] * NBUF
            + [pltpu.VMEM((CB, NCAT, D), jnp.float32)] * NBUF
            + [pltpu.SemaphoreType.DMA] * NBUF
            + [pltpu.SemaphoreType.DMA] * NBUF
            + [pltpu.SemaphoreType.DMA] * NBUF
        ),
    )
    def gather(blk_hbm, tab3_hbm, out_hbm, *refs):
        idx_v = refs[0:NBUF]
        stg_v = refs[NBUF:2 * NBUF]
        isem = refs[2 * NBUF:3 * NBUF]
        gsem = refs[3 * NBUF:4 * NBUF]
        wsem = refs[4 * NBUF:5 * NBUF]

        wid = lax.axis_index("s") * NC + lax.axis_index("c")
        base = wid * RW

        def load_inputs(g, b):
            pltpu.async_copy(blk_hbm.at[wid, g], idx_v[b], isem[b])

        def wait_inputs(g, b):
            pltpu.make_async_copy(blk_hbm.at[wid, g], idx_v[b],
                                  isem[b]).wait()

        def fire_gathers(b):
            def row(r, c):
                v0 = idx_v[b][r, pl.ds(0, L)]
                v1 = idx_v[b][r, pl.ds(L, L)]
                for i in range(NCAT):
                    rowid = v0[i] if i < L else v1[i - L]
                    pltpu.async_copy(
                        tab3_hbm.at[i, pl.ds(rowid, 1), :],
                        stg_v[b].at[pl.ds(r, 1), i, :], gsem[b])
                return c

            lax.fori_loop(0, CB, row, 0)

        def drain(b):
            def row(r, c):
                for i in range(NCAT):
                    pltpu.make_async_copy(
                        tab3_hbm.at[i, pl.ds(0, 1), :],
                        stg_v[b].at[pl.ds(r, 1), i, :], gsem[b]).wait()
                return c

            lax.fori_loop(0, CB, row, 0)

        def issue_writeback(g, b):
            b0 = base + g * CB
            pltpu.async_copy(stg_v[b], out_hbm.at[pl.ds(b0, CB)], wsem[b])

        def wait_writeback(g, b):
            b0 = base + g * CB
            pltpu.make_async_copy(stg_v[b], out_hbm.at[pl.ds(b0, CB)],
                                  wsem[b]).wait()

        for b in range(NBUF - 1):
            load_inputs(b, b)
        wait_inputs(0, 0)
        fire_gathers(0)

        def ring(h, carry):
            g0 = h * NBUF
            for b in range(NBUF):
                g = g0 + b
                gn = g + 1
                bn = (b + 1) % NBUF

                @pl.when(gn < NCHUNK)
                def _(gn=gn, bn=bn):
                    @pl.when(gn >= NBUF)
                    def _():
                        wait_writeback(gn - NBUF, bn)

                    wait_inputs(gn, bn)
                    fire_gathers(bn)

                @pl.when(g + NBUF - 1 < NCHUNK)
                def _(b=b, g=g):
                    load_inputs(g + NBUF - 1, (b + NBUF - 1) % NBUF)

                drain(b)
                issue_writeback(g, b)
            return carry

        lax.fori_loop(0, NCHUNK // NBUF, ring, 0)
        for b in range(NBUF):
            wait_writeback(NCHUNK - NBUF + b, b)

    return gather


def _assemble_body(NCAT, NNUM, D, BC, glin_ref, xnum_ref, pos_ref, w_ref,
                   add_ref, cls_ref, out_ref):
    cat = glin_ref[...] + pos_ref[...][None, 1:1 + NCAT, :]
    num = (xnum_ref[...][:, :, None] * w_ref[...][None]
           + add_ref[...][None])
    clsrow = jnp.broadcast_to(cls_ref[...][None], (BC, 1, D))
    out_ref[...] = jnp.concatenate([clsrow, cat, num], axis=1)


@functools.lru_cache(maxsize=None)
def _build_assemble(B, NCAT, NNUM, D):
    BC = 512
    NTOK = 1 + NCAT + NNUM
    GR = NCAT // 2
    body = functools.partial(_assemble_body, NCAT, NNUM, D, BC)
    return pl.pallas_call(
        body,
        grid=(B // BC,),
        in_specs=[
            pl.BlockSpec((BC, NCAT, D), lambda i: (i, 0, 0)),
            pl.BlockSpec((BC, NNUM), lambda i: (i, 0)),
            pl.BlockSpec((NTOK, D), lambda i: (0, 0)),
            pl.BlockSpec((NNUM, D), lambda i: (0, 0)),
            pl.BlockSpec((NNUM, D), lambda i: (0, 0)),
            pl.BlockSpec((1, D), lambda i: (0, 0)),
        ],
        out_specs=pl.BlockSpec((BC, NTOK, D), lambda i: (i, 0, 0)),
        out_shape=jax.ShapeDtypeStruct((B, NTOK, D), jnp.float32),
    )


def kernel(x_cat, x_num, cat_tables, num_w, num_b, feature_pos, cls):
    B, NCAT = x_cat.shape
    NNUM = x_num.shape[1]
    VROWS, D = cat_tables.shape[1], cat_tables.shape[2]
    NW = 32
    NCHUNK = B // (NW * CB)
    xi = x_cat.astype(jnp.int32).reshape(NW, NCHUNK, CB, NCAT)
    blk = (jnp.zeros((NW, NCHUNK, CB, 128), jnp.int32)
           .at[:, :, :, :NCAT].set(xi))
    glin = _build_gather(B, NCAT, VROWS, D)(blk, cat_tables)
    fn = _build_assemble(B, NCAT, NNUM, D)
    return fn(glin, x_num.astype(jnp.float32), feature_pos, num_w,
              num_b + feature_pos[1 + NCAT:],
              (cls.reshape(1, D) + feature_pos[0:1]))


# speedup vs baseline: 1.3312x; 1.1284x over previous
import functools

import jax
import jax.numpy as jnp
from jax import lax
from jax.experimental import pallas as pl
from jax.experimental.pallas import tpu as pltpu
from jax.experimental.pallas import tpu_sc as plsc

L = 16
NBUF = 4
CB = 4


@functools.lru_cache(maxsize=None)
def _build_gather(B, NCAT, VROWS, D):
    info = plsc.get_sparse_core_info()
    NC, NS = info.num_cores, info.num_subcores
    NW = NC * NS
    GR = NCAT // 2
    RW = B // NW
    NCHUNK = RW // CB
    assert B % (NW * CB) == 0 and NCAT % 2 == 0 and NCHUNK % NBUF == 0

    mesh = plsc.VectorSubcoreMesh(core_axis_name="c", subcore_axis_name="s")

    @functools.partial(
        pl.kernel,
        out_type=jax.ShapeDtypeStruct((B, NCAT, D), jnp.float32),
        mesh=mesh,
        scratch_types=(
            [pltpu.VMEM((CB, 128), jnp.int32)] * NBUF
            + [pltpu.VMEM((CB, NCAT, D), jnp.float32)] * NBUF
            + [pltpu.SemaphoreType.DMA] * NBUF
            + [pltpu.SemaphoreType.DMA] * NBUF
            + [pltpu.SemaphoreType.DMA] * NBUF
        ),
    )
    def gather(blk_hbm, tab3_hbm, out_hbm, *refs):
        idx_v = refs[0:NBUF]
        stg_v = refs[NBUF:2 * NBUF]
        isem = refs[2 * NBUF:3 * NBUF]
        gsem = refs[3 * NBUF:4 * NBUF]
        wsem = refs[4 * NBUF:5 * NBUF]

        wid = lax.axis_index("s") * NC + lax.axis_index("c")
        base = wid * RW

        def load_inputs(g, b):
            pltpu.async_copy(blk_hbm.at[wid, g], idx_v[b], isem[b])

        def wait_inputs(g, b):
            pltpu.make_async_copy(blk_hbm.at[wid, g], idx_v[b],
                                  isem[b]).wait()

        def fire_gathers(b):
            def row(r, c):
                v0 = idx_v[b][r, pl.ds(0, L)]
                v1 = idx_v[b][r, pl.ds(L, L)]
                for i in range(NCAT):
                    rowid = v0[i] if i < L else v1[i - L]
                    pltpu.async_copy(
                        tab3_hbm.at[i, pl.ds(rowid, 1), :],
                        stg_v[b].at[pl.ds(r, 1), i, :], gsem[b])
                return c

            lax.fori_loop(0, CB, row, 0)

        def drain(b):
            def row(r, c):
                for i in range(NCAT):
                    pltpu.make_async_copy(
                        tab3_hbm.at[i, pl.ds(0, 1), :],
                        stg_v[b].at[pl.ds(r, 1), i, :], gsem[b]).wait()
                return c

            lax.fori_loop(0, CB, row, 0)

        def issue_writeback(g, b):
            b0 = base + g * CB
            pltpu.async_copy(stg_v[b], out_hbm.at[pl.ds(b0, CB)], wsem[b])

        def wait_writeback(g, b):
            b0 = base + g * CB
            pltpu.make_async_copy(stg_v[b], out_hbm.at[pl.ds(b0, CB)],
                                  wsem[b]).wait()

        for b in range(NBUF - 1):
            load_inputs(b, b)
        wait_inputs(0, 0)
        fire_gathers(0)

        def ring(h, carry):
            g0 = h * NBUF
            for b in range(NBUF):
                g = g0 + b
                gn = g + 1
                bn = (b + 1) % NBUF

                @pl.when(gn < NCHUNK)
                def _(gn=gn, bn=bn):
                    @pl.when(gn >= NBUF)
                    def _():
                        wait_writeback(gn - NBUF, bn)

                    wait_inputs(gn, bn)
                    fire_gathers(bn)

                @pl.when(g + NBUF - 1 < NCHUNK)
                def _(b=b, g=g):
                    load_inputs(g + NBUF - 1, (b + NBUF - 1) % NBUF)

                drain(b)
                issue_writeback(g, b)
            return carry

        lax.fori_loop(0, NCHUNK // NBUF, ring, 0)
        for b in range(NBUF):
            wait_writeback(NCHUNK - NBUF + b, b)

    return gather


def _assemble_body(NCAT, NNUM, D, BC, glin_ref, xnumt_ref, pos_ref, w_ref,
                   add_ref, cls_ref, out_ref):
    cat = glin_ref[...]
    pos = pos_ref[...]
    out_ref[0] = jnp.broadcast_to(cls_ref[...][0][:, None], (D, BC))
    for t in range(NCAT):
        out_ref[1 + t] = jnp.transpose(cat[:, t, :]) + pos[1 + t][:, None]
    xt = xnumt_ref[...]
    w = w_ref[...]
    add = add_ref[...]
    for j in range(NNUM):
        out_ref[1 + NCAT + j] = w[j][:, None] * xt[j][None, :] + add[j][:, None]


@functools.lru_cache(maxsize=None)
def _build_assemble(B, NCAT, NNUM, D):
    BC = 512
    NTOK = 1 + NCAT + NNUM
    body = functools.partial(_assemble_body, NCAT, NNUM, D, BC)
    return pl.pallas_call(
        body,
        grid=(B // BC,),
        in_specs=[
            pl.BlockSpec((BC, NCAT, D), lambda i: (i, 0, 0)),
            pl.BlockSpec((NNUM, BC), lambda i: (0, i)),
            pl.BlockSpec((NTOK, D), lambda i: (0, 0)),
            pl.BlockSpec((NNUM, D), lambda i: (0, 0)),
            pl.BlockSpec((NNUM, D), lambda i: (0, 0)),
            pl.BlockSpec((1, D), lambda i: (0, 0)),
        ],
        out_specs=pl.BlockSpec((NTOK, D, BC), lambda i: (0, 0, i)),
        out_shape=jax.ShapeDtypeStruct((NTOK, D, B), jnp.float32),
    )


def kernel(x_cat, x_num, cat_tables, num_w, num_b, feature_pos, cls):
    B, NCAT = x_cat.shape
    NNUM = x_num.shape[1]
    VROWS, D = cat_tables.shape[1], cat_tables.shape[2]
    NW = 32
    NCHUNK = B // (NW * CB)
    xi = x_cat.astype(jnp.int32).reshape(NW, NCHUNK, CB, NCAT)
    blk = (jnp.zeros((NW, NCHUNK, CB, 128), jnp.int32)
           .at[:, :, :, :NCAT].set(xi))
    glin = _build_gather(B, NCAT, VROWS, D)(blk, cat_tables)
    fn = _build_assemble(B, NCAT, NNUM, D)
    out_t = fn(glin, x_num.astype(jnp.float32).T, feature_pos, num_w,
               num_b + feature_pos[1 + NCAT:],
               (cls.reshape(1, D) + feature_pos[0:1]))
    return out_t.transpose(2, 0, 1)
